# baseline (speedup 1.0000x reference)
import functools

import jax
import jax.numpy as jnp
from jax import lax
from jax.experimental import pallas as pl
from jax.experimental.pallas import tpu as pltpu
from jax.experimental.pallas import tpu_sc as plsc

B, T, K = 1024, 20, 100000
NC, NS, L = 2, 16, 16
NW = NC * NS
RPW = B // NW
NG = RPW // L
GE = L * T
GEP = 384
RPL = 16
NLP = K // RPL
NSTR = 4
STRW = K // NSTR
SLOTS = 32
SENT = K
FCH = 384
NFCH = STRW // FCH


def _sig(x):
    return 1.0 / (1.0 + jnp.exp(-x))


def _full(v):
    return jnp.full((L,), v, jnp.int32)


KBLK = 4000


def _tc_sigmoid(x_ref, o_ref):
    o_ref[...] = jnp.broadcast_to(_sig(x_ref[...]), (KBLK, 128))


_sig_call = pl.pallas_call(
    _tc_sigmoid,
    grid=(K // KBLK,),
    in_specs=[pl.BlockSpec((KBLK, 1), lambda i: (i, 0))],
    out_specs=pl.BlockSpec((KBLK, 128), lambda i: (i, 0)),
    out_shape=jax.ShapeDtypeStruct((K, 128), jnp.float32),
)


@functools.lru_cache(maxsize=None)
def _build_recurrence():
  mesh = plsc.VectorSubcoreMesh(
      core_axis_name="c", subcore_axis_name="s", num_cores=NC, num_subcores=NS
  )

  @functools.partial(
      pl.kernel,
      out_type=(
          jax.ShapeDtypeStruct((B, T), jnp.float32),
          jax.ShapeDtypeStruct((B, SLOTS), jnp.int32),
          jax.ShapeDtypeStruct((B, SLOTS), jnp.float32),
      ),
      mesh=mesh,
      compiler_params=pltpu.CompilerParams(needs_layout_passes=False),
      scratch_types=[
          pltpu.VMEM((8, 128), jnp.int32),
          pltpu.VMEM((8, 128), jnp.int32),
          pltpu.VMEM((8, 128), jnp.int32),
          pltpu.VMEM((3, 128), jnp.int32),
          pltpu.VMEM((3, 128), jnp.int32),
          pltpu.VMEM((GEP, 128), jnp.float32),
          pltpu.VMEM((GEP, 128), jnp.float32),
          pltpu.VMEM((RPW, T), jnp.float32),
          pltpu.VMEM((RPW, SLOTS), jnp.int32),
          pltpu.VMEM((RPW, SLOTS), jnp.float32),
          pltpu.VMEM(((T - 1) * L,), jnp.int32),
          pltpu.VMEM(((T - 1) * L,), jnp.float32),
          pltpu.SemaphoreType.DMA,
      ],
  )
  def _recur(pk_hbm, ck_hbm, corr_hbm, tab_hbm,
             probs_hbm, ucols_hbm, uvals_hbm,
             pk_v, ck_v, corr_v, lpk_v, lck_v, gpk_v, gck_v,
             probs_v, ucols_v, uvals_v, hpk_v, hval_v, gsem):
      wid = lax.axis_index("s") * NC + lax.axis_index("c")
      base = wid * RPW

      pltpu.sync_copy(pk_hbm.at[pl.ds(wid * 8, 8)], pk_v)
      pltpu.sync_copy(ck_hbm.at[pl.ds(wid * 8, 8)], ck_v)
      pltpu.sync_copy(corr_hbm.at[pl.ds(wid * 8, 8)], corr_v)

      lane = lax.broadcasted_iota(jnp.int32, (L,), 0)

      def ld2(ref, flat):
          return plsc.load_gather(ref, [flat >> 7, flat & 127])

      def st2(ref, flat, x):
          plsc.store_scatter(ref, [flat >> 7, flat & 127], x)

      def sent_body(q, _):
          f = q * L + lane
          plsc.store_scatter(
              ucols_v, [f >> 5, f & (SLOTS - 1)], _full(SENT))
          return _

      lax.fori_loop(0, RPW * SLOTS // L, sent_body, None)

      for g in range(NG):
          def line_body(q, _):
              e = g * GE + q * L + lane
              st2(lpk_v, q * L + lane, ld2(pk_v, e) >> 4)
              st2(lck_v, q * L + lane, ld2(ck_v, e) >> 4)
              return _

          lax.fori_loop(0, GEP // L, line_body, None)
          gds = []
          for c in range(3):
              gds.append(pltpu.async_copy(
                  tab_hbm.at[lpk_v.at[c]],
                  gpk_v.at[pl.ds(c * 128, 128)], gsem))
              gds.append(pltpu.async_copy(
                  tab_hbm.at[lck_v.at[c]],
                  gck_v.at[pl.ds(c * 128, 128)], gsem))
          for d in gds:
              d.wait()

          def pkcol(pk_val, eloc, col):
              return plsc.load_gather(
                  gpk_v, [eloc, ((pk_val & 15) << 3) + col])

          def ckcol(ck_val, eloc, col):
              return plsc.load_gather(
                  gck_v, [eloc, ((ck_val & 15) << 3) + col])

          lrow = g * L + lane
          e0 = lane * T
          ck0 = ld2(ck_v, lrow * T)
          c2 = _sig(ckcol(ck0, e0, 2))
          c3 = _sig(ckcol(ck0, e0, 3))
          cs = _sig(ckcol(ck0, e0, 4))
          plsc.store_scatter(probs_v, [lrow, _full(0)],
                             c2 * (1.0 - cs) + c3 * cs)

          def hist_scan(lo, hi, key, default):
              acc = default
              for j in range(lo, hi):
                  pk_j = hpk_v[pl.ds(j * L, L)]
                  v_j = hval_v[pl.ds(j * L, L)]
                  acc = jnp.where(pk_j == key, v_j, acc)
              return acc

          for i in range(1, T):
              eloc = lane * T + i
              eglob = lrow * T + i
              pk_i = ld2(pk_v, eglob)
              ck_i = ld2(ck_v, eglob)
              corr_i = ld2(corr_v, eglob)
              p0 = _sig(pkcol(pk_i, eloc, 0))
              p1 = _sig(pkcol(pk_i, eloc, 1))
              p2 = _sig(pkcol(pk_i, eloc, 2))
              p3 = _sig(pkcol(pk_i, eloc, 3))
              ss = hist_scan(0, i - 1, pk_i, _sig(pkcol(pk_i, eloc, 4)))
              corrb = corr_i == 1
              po0 = jnp.where(corrb, p2, 1.0 - p2)
              po1 = jnp.where(corrb, p3, 1.0 - p3)
              filt = po1 * ss / (po0 * (1.0 - ss) + po1 * ss)
              pred = p0 * (1.0 - filt) + (1.0 - p1) * filt
              hpk_v[pl.ds((i - 1) * L, L)] = pk_i
              hval_v[pl.ds((i - 1) * L, L)] = pred
              c2 = _sig(ckcol(ck_i, eloc, 2))
              c3 = _sig(ckcol(ck_i, eloc, 3))
              cs = hist_scan(0, i, ck_i, _sig(ckcol(ck_i, eloc, 4)))
              plsc.store_scatter(probs_v, [lrow, _full(i)],
                                 c2 * (1.0 - cs) + c3 * cs)

          for i in range(T - 1):
              pk_i = hpk_v[pl.ds(i * L, L)]
              fin = hist_scan(i + 1, T - 1, pk_i, hval_v[pl.ds(i * L, L)])
              plsc.store_scatter(ucols_v, [lrow, _full(i)], pk_i)
              plsc.store_scatter(uvals_v, [lrow, _full(i)], fin)

      pltpu.sync_copy(probs_v, probs_hbm.at[pl.ds(base, RPW)])
      pltpu.sync_copy(ucols_v, ucols_hbm.at[pl.ds(base, RPW)])
      pltpu.sync_copy(uvals_v, uvals_hbm.at[pl.ds(base, RPW)])

  return _recur


@functools.lru_cache(maxsize=None)
def _build_fill():
  mesh = plsc.VectorSubcoreMesh(
      core_axis_name="c", subcore_axis_name="s", num_cores=NC, num_subcores=NS
  )

  @functools.partial(
      pl.kernel,
      out_type=jax.ShapeDtypeStruct((K, B), jnp.float32),
      mesh=mesh,
      compiler_params=pltpu.CompilerParams(needs_layout_passes=False),
      scratch_types=[
          pltpu.VMEM((2, FCH, 128), jnp.float32),
          pltpu.VMEM((128, SLOTS), jnp.int32),
          pltpu.VMEM((128, SLOTS), jnp.float32),
          pltpu.SemaphoreType.DMA,
          pltpu.SemaphoreType.DMA,
      ],
  )
  def _fill(initt_hbm, ucols_hbm, uvals_hbm, state_hbm,
            buf_v, ucols_v, uvals_v, rsem, wsem):
      wid = lax.axis_index("s") * NC + lax.axis_index("c")
      bj = wid >> 2
      s = wid & 3
      sbase = s * STRW
      TAIL = STRW % FCH

      pltpu.sync_copy(ucols_hbm.at[pl.ds(bj * 128, 128)], ucols_v)
      pltpu.sync_copy(uvals_hbm.at[pl.ds(bj * 128, 128)], uvals_v)

      lane = lax.broadcasted_iota(jnp.int32, (L,), 0)

      def apply_edits(p, kc0, ch):
          def ed_body(q, _):
              f = q * L + lane
              row = f >> 5
              slot = f & (SLOTS - 1)
              kc = plsc.load_gather(ucols_v, [row, slot])
              va = plsc.load_gather(uvals_v, [row, slot])
              m = (kc >= kc0) & (kc < kc0 + ch)
              plsc.store_scatter(
                  buf_v, [jnp.full((L,), p, jnp.int32), kc - kc0, row],
                  va, mask=m)
              return _

          lax.fori_loop(0, 128 * SLOTS // L, ed_body, None)

      def rd_src(kc0, ch):
          return initt_hbm.at[pl.ds(kc0, ch), pl.ds(0, 128)]

      def wr_dst(kc0, ch):
          return state_hbm.at[pl.ds(kc0, ch), pl.ds(bj * 128, 128)]

      pltpu.async_copy(rd_src(sbase, FCH), buf_v.at[0], rsem)

      def chunk_body(q, _):
          p = q & 1
          kc0 = pl.multiple_of(sbase + q * FCH, 8)

          @pl.when(q > 0)
          def _():
              pltpu.make_async_copy(
                  rd_src(sbase, FCH), buf_v.at[1 - p], wsem).wait()

          @pl.when(q + 1 < NFCH)
          def _():
              pltpu.async_copy(
                  rd_src(pl.multiple_of(sbase + (q + 1) * FCH, 8), FCH),
                  buf_v.at[1 - p], rsem)

          pltpu.make_async_copy(rd_src(sbase, FCH), buf_v.at[p], rsem).wait()
          apply_edits(p, kc0, FCH)
          pltpu.async_copy(buf_v.at[p], wr_dst(kc0, FCH), wsem)
          return _

      lax.fori_loop(0, NFCH, chunk_body, None)

      lastp = (NFCH - 1) & 1
      pltpu.make_async_copy(rd_src(sbase, FCH), buf_v.at[lastp], wsem).wait()
      tbase = sbase + NFCH * FCH
      pltpu.sync_copy(rd_src(tbase, TAIL), buf_v.at[0, pl.ds(0, TAIL)])
      apply_edits(0, tbase, TAIL)
      pltpu.sync_copy(buf_v.at[0, pl.ds(0, TAIL)], wr_dst(tbase, TAIL))

  return _fill


def kernel(prev_kc, curr_kc, prev_corr, kc_logits):
    lg = kc_logits.astype(jnp.float32)
    tab = jnp.pad(lg, ((0, 0), (0, 3))).reshape(NLP, 128)
    initt = _sig_call(lg[:, 4].reshape(K, 1))

    def _prep(a):
        a = a.astype(jnp.int32).reshape(NW, RPW * T)
        a = jnp.pad(a, ((0, 0), (0, 8 * 128 - RPW * T)))
        return a.reshape(NW * 8, 128)

    probs, ucols, uvals = _build_recurrence()(
        _prep(prev_kc), _prep(curr_kc), _prep(prev_corr), tab)
    state_t = _build_fill()(initt, ucols, uvals)
    return probs, state_t.T


# speedup vs baseline: 1.7810x; 1.7810x over previous
import functools

import jax
import jax.numpy as jnp
from jax import lax
from jax.experimental import pallas as pl
from jax.experimental.pallas import tpu as pltpu
from jax.experimental.pallas import tpu_sc as plsc

B, T, K = 1024, 20, 100000
NC, NS, L = 2, 16, 16
NW = NC * NS
RPW = B // NW
NG = RPW // L
NIDX = RPW * T
RPL = 16
NLP = K // RPL
NSTR = 4
STRW = K // NSTR
SLOTS = 32
SENT = K
FCH = 384
NFCH = STRW // FCH


def _sig(x):
    return 1.0 / (1.0 + jnp.exp(-x))


def _full(v):
    return jnp.full((L,), v, jnp.int32)


KPAD = 100096
KBLK = 5888


def _tc_sigmoid(x_ref, o_ref):
    o_ref[...] = jnp.broadcast_to(
        _sig(x_ref[...]).reshape(KBLK, 1), (KBLK, 128))


_sig_call = pl.pallas_call(
    _tc_sigmoid,
    grid=(KPAD // KBLK,),
    in_specs=[pl.BlockSpec((1, KBLK), lambda i: (0, i))],
    out_specs=pl.BlockSpec((KBLK, 128), lambda i: (i, 0)),
    out_shape=jax.ShapeDtypeStruct((KPAD, 128), jnp.float32),
)


@functools.lru_cache(maxsize=None)
def _build_recurrence():
  mesh = plsc.VectorSubcoreMesh(
      core_axis_name="c", subcore_axis_name="s", num_cores=NC, num_subcores=NS
  )

  @functools.partial(
      pl.kernel,
      out_type=(
          jax.ShapeDtypeStruct((B, T), jnp.float32),
          jax.ShapeDtypeStruct((B, SLOTS), jnp.int32),
          jax.ShapeDtypeStruct((B, SLOTS), jnp.float32),
      ),
      mesh=mesh,
      compiler_params=pltpu.CompilerParams(needs_layout_passes=False),
      scratch_types=[
          pltpu.VMEM((8, 128), jnp.int32),
          pltpu.VMEM((8, 128), jnp.int32),
          pltpu.VMEM((8, 128), jnp.int32),
          pltpu.VMEM((NIDX * 8 // 128, 128), jnp.int32),
          pltpu.VMEM((NIDX * 8 // 128, 128), jnp.float32),
          pltpu.VMEM((RPW, T), jnp.float32),
          pltpu.VMEM((RPW, SLOTS), jnp.int32),
          pltpu.VMEM((RPW, SLOTS), jnp.float32),
          pltpu.VMEM(((T - 1) * L,), jnp.int32),
          pltpu.VMEM(((T - 1) * L,), jnp.float32),
          pltpu.SemaphoreType.DMA,
      ],
  )
  def _recur(pk_hbm, ck_hbm, corr_hbm, tab_hbm,
             probs_hbm, ucols_hbm, uvals_hbm,
             pk_v, ck_v, corr_v, idx_v, gval_v,
             probs_v, ucols_v, uvals_v, hpk_v, hval_v, gsem):
      wid = lax.axis_index("s") * NC + lax.axis_index("c")
      base = wid * RPW

      pltpu.sync_copy(pk_hbm.at[pl.ds(wid * 8, 8)], pk_v)
      pltpu.sync_copy(ck_hbm.at[pl.ds(wid * 8, 8)], ck_v)
      pltpu.sync_copy(corr_hbm.at[pl.ds(wid * 8, 8)], corr_v)

      lane = lax.broadcasted_iota(jnp.int32, (L,), 0)

      def ld2(ref, flat):
          return plsc.load_gather(ref, [flat >> 7, flat & 127])

      def st2(ref, flat, x):
          plsc.store_scatter(ref, [flat >> 7, flat & 127], x)

      def sent_body(q, _):
          f = q * L + lane
          plsc.store_scatter(
              ucols_v, [f >> 5, f & (SLOTS - 1)], _full(SENT))
          return _

      lax.fori_loop(0, RPW * SLOTS // L, sent_body, None)

      def idx_body(q, _):
          e = q * L + lane
          pkq = ld2(pk_v, e)
          ckq = ld2(ck_v, e)
          for c in range(5):
              st2(idx_v, e * 8 + c, c * K + pkq)
          for c in range(3):
              st2(idx_v, e * 8 + 5 + c, (2 + c) * K + ckq)
          return _

      lax.fori_loop(0, NIDX // L, idx_body, None)

      gds = [
          pltpu.async_copy(tab_hbm.at[idx_v.at[j]], gval_v.at[j], gsem)
          for j in range(NIDX * 8 // 128)
      ]
      for d in gds:
          d.wait()

      def gv(e, slot):
          return ld2(gval_v, e * 8 + slot)

      for g in range(NG):
          lrow = g * L + lane
          e0 = lrow * T
          c2 = _sig(gv(e0, 5))
          c3 = _sig(gv(e0, 6))
          cs = _sig(gv(e0, 7))
          plsc.store_scatter(probs_v, [lrow, _full(0)],
                             c2 * (1.0 - cs) + c3 * cs)

          def hist_scan(lo, hi, key, default):
              acc = default
              for j in range(lo, hi):
                  pk_j = hpk_v[pl.ds(j * L, L)]
                  v_j = hval_v[pl.ds(j * L, L)]
                  acc = jnp.where(pk_j == key, v_j, acc)
              return acc

          for i in range(1, T):
              eglob = lrow * T + i
              pk_i = ld2(pk_v, eglob)
              ck_i = ld2(ck_v, eglob)
              corr_i = ld2(corr_v, eglob)
              p0 = _sig(gv(eglob, 0))
              p1 = _sig(gv(eglob, 1))
              p2 = _sig(gv(eglob, 2))
              p3 = _sig(gv(eglob, 3))
              ss = hist_scan(0, i - 1, pk_i, _sig(gv(eglob, 4)))
              corrb = corr_i == 1
              po0 = jnp.where(corrb, p2, 1.0 - p2)
              po1 = jnp.where(corrb, p3, 1.0 - p3)
              filt = po1 * ss / (po0 * (1.0 - ss) + po1 * ss)
              pred = p0 * (1.0 - filt) + (1.0 - p1) * filt
              hpk_v[pl.ds((i - 1) * L, L)] = pk_i
              hval_v[pl.ds((i - 1) * L, L)] = pred
              c2 = _sig(gv(eglob, 5))
              c3 = _sig(gv(eglob, 6))
              cs = hist_scan(0, i, ck_i, _sig(gv(eglob, 7)))
              plsc.store_scatter(probs_v, [lrow, _full(i)],
                                 c2 * (1.0 - cs) + c3 * cs)

          for i in range(T - 1):
              pk_i = hpk_v[pl.ds(i * L, L)]
              fin = hist_scan(i + 1, T - 1, pk_i, hval_v[pl.ds(i * L, L)])
              plsc.store_scatter(ucols_v, [lrow, _full(i)], pk_i)
              plsc.store_scatter(uvals_v, [lrow, _full(i)], fin)

      pltpu.sync_copy(probs_v, probs_hbm.at[pl.ds(base, RPW)])
      pltpu.sync_copy(ucols_v, ucols_hbm.at[pl.ds(base, RPW)])
      pltpu.sync_copy(uvals_v, uvals_hbm.at[pl.ds(base, RPW)])

  return _recur


@functools.lru_cache(maxsize=None)
def _build_fill():
  mesh = plsc.VectorSubcoreMesh(
      core_axis_name="c", subcore_axis_name="s", num_cores=NC, num_subcores=NS
  )

  @functools.partial(
      pl.kernel,
      out_type=jax.ShapeDtypeStruct((K, B), jnp.float32),
      mesh=mesh,
      compiler_params=pltpu.CompilerParams(needs_layout_passes=False),
      scratch_types=[
          pltpu.VMEM((2, FCH, 128), jnp.float32),
          pltpu.VMEM((128, SLOTS), jnp.int32),
          pltpu.VMEM((128, SLOTS), jnp.float32),
          pltpu.SemaphoreType.DMA,
          pltpu.SemaphoreType.DMA,
      ],
  )
  def _fill(initt_hbm, ucols_hbm, uvals_hbm, state_hbm,
            buf_v, ucols_v, uvals_v, rsem, wsem):
      wid = lax.axis_index("s") * NC + lax.axis_index("c")
      bj = wid >> 2
      s = wid & 3
      sbase = s * STRW
      TAIL = STRW % FCH

      pltpu.sync_copy(ucols_hbm.at[pl.ds(bj * 128, 128)], ucols_v)
      pltpu.sync_copy(uvals_hbm.at[pl.ds(bj * 128, 128)], uvals_v)

      lane = lax.broadcasted_iota(jnp.int32, (L,), 0)

      def apply_edits(p, kc0, ch):
          def ed_body(q, _):
              f = q * L + lane
              row = f >> 5
              slot = f & (SLOTS - 1)
              kc = plsc.load_gather(ucols_v, [row, slot])
              va = plsc.load_gather(uvals_v, [row, slot])
              m = (kc >= kc0) & (kc < kc0 + ch)
              plsc.store_scatter(
                  buf_v, [jnp.full((L,), p, jnp.int32), kc - kc0, row],
                  va, mask=m)
              return _

          lax.fori_loop(0, 128 * SLOTS // L, ed_body, None)

      def rd_src(kc0, ch):
          return initt_hbm.at[pl.ds(kc0, ch), pl.ds(0, 128)]

      def wr_dst(kc0, ch):
          return state_hbm.at[pl.ds(kc0, ch), pl.ds(bj * 128, 128)]

      pltpu.async_copy(rd_src(sbase, FCH), buf_v.at[0], rsem)

      def chunk_body(q, _):
          p = q & 1
          kc0 = pl.multiple_of(sbase + q * FCH, 8)

          @pl.when(q > 0)
          def _():
              pltpu.make_async_copy(
                  rd_src(sbase, FCH), buf_v.at[1 - p], wsem).wait()

          @pl.when(q + 1 < NFCH)
          def _():
              pltpu.async_copy(
                  rd_src(pl.multiple_of(sbase + (q + 1) * FCH, 8), FCH),
                  buf_v.at[1 - p], rsem)

          pltpu.make_async_copy(rd_src(sbase, FCH), buf_v.at[p], rsem).wait()
          apply_edits(p, kc0, FCH)
          pltpu.async_copy(buf_v.at[p], wr_dst(kc0, FCH), wsem)
          return _

      lax.fori_loop(0, NFCH, chunk_body, None)

      lastp = (NFCH - 1) & 1
      pltpu.make_async_copy(rd_src(sbase, FCH), buf_v.at[lastp], wsem).wait()
      tbase = sbase + NFCH * FCH
      pltpu.sync_copy(rd_src(tbase, TAIL), buf_v.at[0, pl.ds(0, TAIL)])
      apply_edits(0, tbase, TAIL)
      pltpu.sync_copy(buf_v.at[0, pl.ds(0, TAIL)], wr_dst(tbase, TAIL))

  return _fill


def kernel(prev_kc, curr_kc, prev_corr, kc_logits):
    lg = kc_logits.astype(jnp.float32)
    tab = lg.T.reshape(5 * K)
    initt = _sig_call(jnp.pad(lg[:, 4], (0, KPAD - K)).reshape(1, KPAD))

    def _prep(a):
        a = a.astype(jnp.int32).reshape(NW, RPW * T)
        a = jnp.pad(a, ((0, 0), (0, 8 * 128 - RPW * T)))
        return a.reshape(NW * 8, 128)

    probs, ucols, uvals = _build_recurrence()(
        _prep(prev_kc), _prep(curr_kc), _prep(prev_corr), tab)
    state_t = _build_fill()(initt, ucols, uvals)
    return probs, state_t.T
